# baseline (speedup 1.0000x reference)
import functools

import jax
import jax.numpy as jnp
import numpy as np
from jax import lax
from jax.experimental import pallas as pl
from jax.experimental.pallas import tpu as pltpu
from jax.experimental.pallas import tpu_sc as plsc

_B = 8
_NI = 2048
_NS = 512
_BJ = 128


def _sort_key(x):
    i = lax.bitcast_convert_type(x, jnp.int32)
    return i ^ ((i >> 31) & jnp.int32(0x7FFFFFFF))


def _noised(s, sf, m, nz):
    x = jnp.where(m > 0, jnp.maximum(s, sf), s)
    return jnp.clip(x, -1.0, 1.0) + nz


def _tri_mask():
    return (np.arange(_BJ)[:, None] < np.arange(_BJ)[None, :]).astype(np.int32)


_RP = 8


def _rank_one_row(ki, kT, tri):
    shr = lax.shift_right_logical
    acc = jnp.zeros((_BJ, _NI), jnp.int32)
    for jc in range(_NI // _BJ):
        lo = jc * _BJ
        kj = kT[:, jc:jc + 1]
        kj1 = kj - 1
        parts = []
        if lo > 0:
            parts.append(shr(kj - ki[:, :lo], 31))
        parts.append(shr(kj - ki[:, lo:lo + _BJ] - tri, 31))
        if lo + _BJ < _NI:
            parts.append(shr(kj1 - ki[:, lo + _BJ:], 31))
        acc += jnp.concatenate(parts, axis=1)
    return jnp.sum(acc, axis=0, keepdims=True)


def _rank_body(n_ref, s_r, sf_r, m_r, nz_r, s_c, sf_c, m_c, nz_c, g_ref,
               rank_ref, w_ref):
    tri = g_ref[...]
    for r in range(_RP):
        xi = _noised(s_r[r], sf_r[r], m_r[r], nz_r[r])
        xc = _noised(s_c[r], sf_c[r], m_c[r], nz_c[r])
        ki = _sort_key(xi)
        kT = jnp.transpose(_sort_key(xc))
        rank_ref[r] = _rank_one_row(ki, kT, tri)

    nv = n_ref[0]
    pos = lax.broadcasted_iota(jnp.int32, (1, _NS), 1).astype(jnp.float32)
    dr = pos + float(_NI - _NS)
    w_ref[...] = jnp.broadcast_to(
        jnp.minimum((dr - (float(_NI) - nv)) / (0.1 * nv), 1.0), (_RP, 1, _NS))


def _tc_rank(n_arr, s, sflip, maskf, noise):
    nc = _NI // _BJ
    row = pl.BlockSpec((_RP, 1, _NI), lambda b: (b, 0, 0))
    col = pl.BlockSpec((_RP, nc, _BJ), lambda b: (b, 0, 0))
    sr, sfr, mr, nzr = (a.reshape(_B, 1, _NI) for a in (s, sflip, maskf, noise))
    s3, sf3, m3, nz3 = (a.reshape(_B, nc, _BJ) for a in (s, sflip, maskf, noise))
    rank3, w3 = pl.pallas_call(
        _rank_body,
        grid=(_B // _RP,),
        in_specs=[pl.BlockSpec(memory_space=pltpu.SMEM),
                  row, row, row, row, col, col, col, col,
                  pl.BlockSpec((_BJ, _BJ), lambda b: (0, 0))],
        out_specs=[pl.BlockSpec((_RP, 1, _NI), lambda b: (b, 0, 0)),
                   pl.BlockSpec((_RP, 1, _NS), lambda b: (b, 0, 0))],
        out_shape=[jax.ShapeDtypeStruct((_B, 1, _NI), jnp.int32),
                   jax.ShapeDtypeStruct((_B, 1, _NS), jnp.float32)],
    )(n_arr, sr, sfr, mr, nzr, s3, sf3, m3, nz3, _tri_mask())
    return rank3.reshape(_B, _NI), w3.reshape(_B, _NS)


@functools.lru_cache(maxsize=None)
def _make_sc_invert():
    mesh = plsc.VectorSubcoreMesh(core_axis_name="c", subcore_axis_name="s",
                                  num_cores=1, num_subcores=_B)

    @functools.partial(
        pl.kernel,
        out_type=jax.ShapeDtypeStruct((_B, _NS), jnp.int32),
        mesh=mesh,
        scratch_types=[pltpu.VMEM((_NI,), jnp.int32),
                       pltpu.VMEM((_NI,), jnp.int32)],
        compiler_params=pltpu.CompilerParams(needs_layout_passes=False),
    )
    def _sc_invert(rank_hbm, out_hbm, rank_v, inv_v):
        cid = lax.axis_index("c")
        sid = lax.axis_index("s")

        @pl.when((cid == 0) & (sid < _B))
        def _():
            pltpu.sync_copy(rank_hbm.at[sid], rank_v)
            base = lax.iota(jnp.int32, 16)
            for k in range(_NI // 16):
                rk = rank_v[pl.ds(k * 16, 16)]
                plsc.store_scatter(inv_v, [rk], base + (k * 16))
            pltpu.sync_copy(inv_v.at[pl.ds(_NI - _NS, _NS)], out_hbm.at[sid])

    return _sc_invert


@functools.lru_cache(maxsize=None)
def _prng_consts():
    with jax.ensure_compile_time_eval():
        k1, k2, k3 = jax.random.split(jax.random.key(42), 3)
        mask = jax.random.uniform(k1, (_B, _NI)) < 0.1
        bmask = jax.random.uniform(k2, (_B, 1)) < 0.75
        maskf = jnp.logical_and(mask, bmask).astype(jnp.float32)
        noise = jax.random.normal(k3, (_B, _NI), dtype=jnp.float32) * (4.0 / _NI)
    return np.asarray(maskf), np.asarray(noise)


def kernel(scores, N):
    maskf, noise = _prng_consts()
    sflip = jnp.flip(scores, (0, 1))
    n_arr = jnp.asarray(N, jnp.float32).reshape(1)
    rank, weight = _tc_rank(n_arr, scores, sflip, maskf, noise)
    idx_keep = _make_sc_invert()(rank)
    return idx_keep, weight


# speedup vs baseline: 1.0042x; 1.0042x over previous
import functools

import jax
import jax.numpy as jnp
import numpy as np
from jax import lax
from jax.experimental import pallas as pl
from jax.experimental.pallas import tpu as pltpu
from jax.experimental.pallas import tpu_sc as plsc

_B = 8
_NI = 2048
_NS = 512
_BJ = 128


def _sort_key(x):
    i = lax.bitcast_convert_type(x, jnp.int32)
    return i ^ ((i >> 31) & jnp.int32(0x7FFFFFFF))


def _noised(s, sf, m, nz):
    x = jnp.where(m > 0, jnp.maximum(s, sf), s)
    return jnp.clip(x, -1.0, 1.0) + nz


def _tri_mask():
    return (np.arange(_BJ)[:, None] < np.arange(_BJ)[None, :]).astype(np.int32)


_RP = 4


def _rank_one_row(ki, kT, tri):
    shr = lax.shift_right_logical
    accs = [jnp.zeros((_BJ, _NI), jnp.int32), jnp.zeros((_BJ, _NI), jnp.int32)]
    for jc in range(_NI // _BJ):
        lo = jc * _BJ
        kj = kT[:, jc:jc + 1]
        kj1 = kj - 1
        parts = []
        if lo > 0:
            parts.append(shr(kj - ki[:, :lo], 31))
        parts.append(shr(kj - ki[:, lo:lo + _BJ] - tri, 31))
        if lo + _BJ < _NI:
            parts.append(shr(kj1 - ki[:, lo + _BJ:], 31))
        accs[jc % 2] += jnp.concatenate(parts, axis=1)
    return jnp.sum(accs[0] + accs[1], axis=0, keepdims=True)


def _rank_body(n_ref, s_r, sf_r, m_r, nz_r, s_c, sf_c, m_c, nz_c, g_ref,
               rank_ref, w_ref):
    tri = g_ref[...]
    for r in range(_RP):
        xi = _noised(s_r[r], sf_r[r], m_r[r], nz_r[r])
        xc = _noised(s_c[r], sf_c[r], m_c[r], nz_c[r])
        ki = _sort_key(xi)
        kT = jnp.transpose(_sort_key(xc))
        rank_ref[r] = _rank_one_row(ki, kT, tri)

    nv = n_ref[0]
    pos = lax.broadcasted_iota(jnp.int32, (1, _NS), 1).astype(jnp.float32)
    dr = pos + float(_NI - _NS)
    w_ref[...] = jnp.broadcast_to(
        jnp.minimum((dr - (float(_NI) - nv)) / (0.1 * nv), 1.0), (_RP, 1, _NS))


def _tc_rank(n_arr, s, sflip, maskf, noise):
    nc = _NI // _BJ
    row = pl.BlockSpec((_RP, 1, _NI), lambda b: (b, 0, 0))
    col = pl.BlockSpec((_RP, nc, _BJ), lambda b: (b, 0, 0))
    sr, sfr, mr, nzr = (a.reshape(_B, 1, _NI) for a in (s, sflip, maskf, noise))
    s3, sf3, m3, nz3 = (a.reshape(_B, nc, _BJ) for a in (s, sflip, maskf, noise))
    rank3, w3 = pl.pallas_call(
        _rank_body,
        grid=(_B // _RP,),
        in_specs=[pl.BlockSpec(memory_space=pltpu.SMEM),
                  row, row, row, row, col, col, col, col,
                  pl.BlockSpec((_BJ, _BJ), lambda b: (0, 0))],
        out_specs=[pl.BlockSpec((_RP, 1, _NI), lambda b: (b, 0, 0)),
                   pl.BlockSpec((_RP, 1, _NS), lambda b: (b, 0, 0))],
        out_shape=[jax.ShapeDtypeStruct((_B, 1, _NI), jnp.int32),
                   jax.ShapeDtypeStruct((_B, 1, _NS), jnp.float32)],
    )(n_arr, sr, sfr, mr, nzr, s3, sf3, m3, nz3, _tri_mask())
    return rank3.reshape(_B, _NI), w3.reshape(_B, _NS)


@functools.lru_cache(maxsize=None)
def _make_sc_invert():
    mesh = plsc.VectorSubcoreMesh(core_axis_name="c", subcore_axis_name="s",
                                  num_cores=1, num_subcores=_B)

    @functools.partial(
        pl.kernel,
        out_type=jax.ShapeDtypeStruct((_B, _NS), jnp.int32),
        mesh=mesh,
        scratch_types=[pltpu.VMEM((_NI,), jnp.int32),
                       pltpu.VMEM((_NI,), jnp.int32)],
        compiler_params=pltpu.CompilerParams(needs_layout_passes=False),
    )
    def _sc_invert(rank_hbm, out_hbm, rank_v, inv_v):
        cid = lax.axis_index("c")
        sid = lax.axis_index("s")

        @pl.when((cid == 0) & (sid < _B))
        def _():
            pltpu.sync_copy(rank_hbm.at[sid], rank_v)
            base = lax.iota(jnp.int32, 16)
            for k in range(_NI // 16):
                rk = rank_v[pl.ds(k * 16, 16)]
                plsc.store_scatter(inv_v, [rk], base + (k * 16))
            pltpu.sync_copy(inv_v.at[pl.ds(_NI - _NS, _NS)], out_hbm.at[sid])

    return _sc_invert


@functools.lru_cache(maxsize=None)
def _prng_consts():
    with jax.ensure_compile_time_eval():
        k1, k2, k3 = jax.random.split(jax.random.key(42), 3)
        mask = jax.random.uniform(k1, (_B, _NI)) < 0.1
        bmask = jax.random.uniform(k2, (_B, 1)) < 0.75
        maskf = jnp.logical_and(mask, bmask).astype(jnp.float32)
        noise = jax.random.normal(k3, (_B, _NI), dtype=jnp.float32) * (4.0 / _NI)
    return np.asarray(maskf), np.asarray(noise)


def kernel(scores, N):
    maskf, noise = _prng_consts()
    sflip = jnp.flip(scores, (0, 1))
    n_arr = jnp.asarray(N, jnp.float32).reshape(1)
    rank, weight = _tc_rank(n_arr, scores, sflip, maskf, noise)
    idx_keep = _make_sc_invert()(rank)
    return idx_keep, weight
